# Initial kernel scaffold; baseline (speedup 1.0000x reference)
import jax
import jax.numpy as jnp
from jax.experimental import pallas as pl


def kernel(indices, weight):
    raise NotImplementedError("write your pallas kernel here")


# speedup vs baseline: 4.6081x; 4.6081x over previous
import functools

import jax
import jax.numpy as jnp
from jax import lax
from jax.experimental import pallas as pl
from jax.experimental.pallas import tpu as pltpu
from jax.experimental.pallas import tpu_sc as plsc

NC = 2
NS = 16
NW = NC * NS


@functools.cache
def _build_lookup(B, V, D, C):
    b_per_w = B // NW
    nchunk = b_per_w // C
    mesh = plsc.VectorSubcoreMesh(core_axis_name="c", subcore_axis_name="s")

    @functools.partial(
        pl.kernel,
        mesh=mesh,
        out_type=jax.ShapeDtypeStruct((B, D), jnp.float32),
        compiler_params=pltpu.CompilerParams(use_tc_tiling_on_sc=False),
        scratch_types=[
            pltpu.VMEM((b_per_w,), jnp.int32),
            pltpu.VMEM((C, D), jnp.float32),
            pltpu.SemaphoreType.DMA,
        ],
    )
    def lookup(idx_hbm, table_hbm, out_hbm, idx_v, rows_v, sem):
        wid = lax.axis_index("s") * NC + lax.axis_index("c")
        base = wid * b_per_w
        pltpu.sync_copy(idx_hbm.at[pl.ds(base, b_per_w)], idx_v)
        for c in range(nchunk):
            pltpu.async_copy(
                table_hbm.at[idx_v.at[pl.ds(c * C, C)]], rows_v, sem
            ).wait()
            pltpu.sync_copy(rows_v, out_hbm.at[pl.ds(base + c * C, C)])

    return lookup


def kernel(indices, weight):
    B = indices.size
    V, D = weight.shape
    idx = indices.reshape(-1).astype(jnp.int32)
    out = _build_lookup(B, V, D, 800)(idx, weight)
    return out.reshape(*indices.shape, D)


# speedup vs baseline: 4.6634x; 1.0120x over previous
import functools

import jax
import jax.numpy as jnp
from jax import lax
from jax.experimental import pallas as pl
from jax.experimental.pallas import tpu as pltpu
from jax.experimental.pallas import tpu_sc as plsc

NC = 2
NS = 16
NW = NC * NS
NBUF = 4


@functools.cache
def _build_lookup(B, V, D, C):
    b_per_w = B // NW
    nchunk = b_per_w // C
    mesh = plsc.VectorSubcoreMesh(core_axis_name="c", subcore_axis_name="s")

    @functools.partial(
        pl.kernel,
        mesh=mesh,
        out_type=jax.ShapeDtypeStruct((B, D), jnp.float32),
        compiler_params=pltpu.CompilerParams(use_tc_tiling_on_sc=False),
        scratch_types=[
            pltpu.VMEM((b_per_w,), jnp.int32),
            pltpu.VMEM((NBUF, C, D), jnp.float32),
            pltpu.SemaphoreType.DMA((NBUF,)),
            pltpu.SemaphoreType.DMA((NBUF,)),
        ],
    )
    def lookup(idx_hbm, table_hbm, out_hbm, idx_v, rows_v, gsem, ssem):
        wid = lax.axis_index("s") * NC + lax.axis_index("c")
        base = wid * b_per_w

        def gather(c):
            b = c % NBUF
            return pltpu.make_async_copy(
                table_hbm.at[idx_v.at[pl.ds(c * C, C)]], rows_v.at[b], gsem.at[b]
            )

        def outcopy(c):
            b = c % NBUF
            return pltpu.make_async_copy(
                rows_v.at[b], out_hbm.at[pl.ds(base + c * C, C)], ssem.at[b]
            )

        pltpu.sync_copy(idx_hbm.at[pl.ds(base, b_per_w)], idx_v)
        for c in range(min(2, nchunk)):
            gather(c).start()
        for c in range(nchunk):
            if c >= 2:
                outcopy(c - 2).wait()
            if c + 2 < nchunk:
                gather(c + 2).start()
            gather(c).wait()
            outcopy(c).start()
        for c in range(max(0, nchunk - 2), nchunk):
            outcopy(c).wait()

    return lookup


def kernel(indices, weight):
    B = indices.size
    V, D = weight.shape
    idx = indices.reshape(-1).astype(jnp.int32)
    out = _build_lookup(B, V, D, 400)(idx, weight)
    return out.reshape(*indices.shape, D)


# speedup vs baseline: 6.2400x; 1.3381x over previous
import functools

import jax
import jax.numpy as jnp
from jax import lax
from jax.experimental import pallas as pl
from jax.experimental.pallas import tpu as pltpu
from jax.experimental.pallas import tpu_sc as plsc

NC = 2
NS = 16
NW = NC * NS
R = 50
D = 64
BL = 128


@functools.cache
def _build_lookup(B, V):
    nb = B // R
    assert nb == NW * BL
    mesh = plsc.VectorSubcoreMesh(core_axis_name="c", subcore_axis_name="s")

    @functools.partial(
        pl.kernel,
        mesh=mesh,
        out_type=jax.ShapeDtypeStruct((R * 8, NW, 8, BL), jnp.float32),
        compiler_params=pltpu.CompilerParams(
            use_tc_tiling_on_sc=False, needs_layout_passes=False
        ),
        scratch_types=[
            pltpu.VMEM((BL * R,), jnp.int32),
            pltpu.VMEM((R, BL), jnp.int32),
            pltpu.VMEM((2, BL, D), jnp.float32),
            pltpu.VMEM((2, 8, 8, 129), jnp.float32),
            pltpu.SemaphoreType.DMA((2,)),
            pltpu.SemaphoreType.DMA((2,)),
        ],
    )
    def lookup(idx_hbm, table_hbm, out_hbm, idx_raw, idx_t, g, t, gsem, wsem):
        wid = lax.axis_index("s") * NC + lax.axis_index("c")
        pltpu.sync_copy(idx_hbm.at[pl.ds(wid * BL * R, BL * R)], idx_raw)

        lanes = jax.lax.iota(jnp.int32, 16)

        def tr_idx(r, _):
            for b0 in range(0, BL, 16):
                iv = (lanes + b0) * R + r
                idx_t[r, pl.ds(b0, 16)] = plsc.load_gather(idx_raw, [iv])
            return 0

        lax.fori_loop(0, R, tr_idx, 0)

        def gather(r, buf):
            return pltpu.make_async_copy(
                table_hbm.at[idx_t.at[r]], g.at[buf], gsem.at[buf]
            )

        def outwrite(r, buf):
            return pltpu.make_async_copy(
                t.at[buf, :, :, pl.ds(0, BL)],
                out_hbm.at[pl.ds(r * 8, 8), wid],
                wsem.at[buf],
            )

        ivs = [((lanes + d0) >> 3, (lanes + d0) & 7) for d0 in range(0, D, 16)]

        gather(0, 0).start()

        def body(r, buf):
            gather(r, buf).wait()

            @pl.when(r + 1 < R)
            def _():
                gather(r + 1, 1 - buf).start()

            @pl.when(r >= 2)
            def _():
                outwrite(r - 2, buf).wait()

            gb = g.at[buf]
            tb = t.at[buf]

            def tr_row(bl, _):
                blv = jnp.broadcast_to(bl, (16,)).astype(jnp.int32)
                for k, (iv0, iv1) in enumerate(ivs):
                    x = gb[bl, pl.ds(k * 16, 16)]
                    plsc.store_scatter(tb, [iv0, iv1, blv], x)
                return 0

            lax.fori_loop(0, BL, tr_row, 0)
            outwrite(r, buf).start()

        def pair(i, _):
            body(2 * i, 0)
            body(2 * i + 1, 1)
            return 0

        lax.fori_loop(0, R // 2, pair, 0)
        outwrite(R - 2, 0).wait()
        outwrite(R - 1, 1).wait()

    return lookup


def kernel(indices, weight):
    B = indices.size
    V, _ = weight.shape
    idx = indices.reshape(-1).astype(jnp.int32)
    out = _build_lookup(B, V)(idx, weight)
    out = out.reshape(R, 8, NW, 8, BL).transpose(2, 4, 0, 1, 3)
    return out.reshape(*indices.shape, D)


# speedup vs baseline: 7.9615x; 1.2759x over previous
import functools

import jax
import jax.numpy as jnp
from jax import lax
from jax.experimental import pallas as pl
from jax.experimental.pallas import tpu as pltpu
from jax.experimental.pallas import tpu_sc as plsc

NC = 2
NS = 16
NW = NC * NS
R = 50
D = 64
BL = 128


@functools.cache
def _build_lookup(B, V):
    nb = B // R
    assert nb == NW * BL
    mesh = plsc.VectorSubcoreMesh(core_axis_name="c", subcore_axis_name="s")

    @functools.partial(
        pl.kernel,
        mesh=mesh,
        out_type=jax.ShapeDtypeStruct((R * 8, NW, 8, BL), jnp.float32),
        compiler_params=pltpu.CompilerParams(
            use_tc_tiling_on_sc=False, needs_layout_passes=False
        ),
        scratch_types=[
            pltpu.VMEM((BL * R,), jnp.int32),
            pltpu.VMEM((R, BL), jnp.int32),
            pltpu.VMEM((BL, D), jnp.float32),
            pltpu.VMEM((BL, D), jnp.float32),
            pltpu.VMEM((D, BL), jnp.float32),
            pltpu.VMEM((D, BL), jnp.float32),
            pltpu.SemaphoreType.DMA((2,)),
            pltpu.SemaphoreType.DMA((2,)),
        ],
    )
    def lookup(
        idx_hbm, table_hbm, out_hbm, idx_raw, idx_t, g0, g1, t0, t1, gsem, wsem
    ):
        gbufs, tbufs = (g0, g1), (t0, t1)
        wid = lax.axis_index("s") * NC + lax.axis_index("c")
        pltpu.sync_copy(idx_hbm.at[pl.ds(wid * BL * R, BL * R)], idx_raw)

        lanes = jax.lax.iota(jnp.int32, 16)

        def tr_idx(r, _):
            for b0 in range(0, BL, 16):
                iv = (lanes + b0) * R + r
                idx_t[r, pl.ds(b0, 16)] = plsc.load_gather(idx_raw, [iv])
            return 0

        lax.fori_loop(0, R, tr_idx, 0)

        def gather(r, buf):
            return pltpu.make_async_copy(
                table_hbm.at[idx_t.at[r]], gbufs[buf], gsem.at[buf]
            )

        def outwrites(r, buf):
            return [
                pltpu.make_async_copy(
                    tbufs[buf].at[pl.ds(dt * 8, 8)],
                    out_hbm.at[r * 8 + dt, wid],
                    wsem.at[buf],
                )
                for dt in range(8)
            ]

        cks = [(lanes + k) & 15 for k in range(16)]

        gather(0, 0).start()

        def body(r, buf):
            gather(r, buf).wait()

            @pl.when(r + 1 < R)
            def _():
                gather(r + 1, 1 - buf).start()

            @pl.when(r >= 2)
            def _():
                for w in outwrites(r - 2, buf):
                    w.wait()

            def tr_blk(i, _):
                ivr = lanes + i * 16
                for d0 in range(0, D, 16):
                    for k4 in range(0, 16, 4):
                        ivcs = [cks[k4 + m] + d0 for m in range(4)]
                        xs = [
                            plsc.load_gather(gbufs[buf], [ivr, ivc])
                            for ivc in ivcs
                        ]
                        for ivc, x in zip(ivcs, xs):
                            plsc.store_scatter(tbufs[buf], [ivc, ivr], x)
                return 0

            lax.fori_loop(0, BL // 16, tr_blk, 0)
            for w in outwrites(r, buf):
                w.start()

        def pair(i, _):
            body(2 * i, 0)
            body(2 * i + 1, 1)
            return 0

        lax.fori_loop(0, R // 2, pair, 0)
        for w in outwrites(R - 2, 0):
            w.wait()
        for w in outwrites(R - 1, 1):
            w.wait()

    return lookup


def kernel(indices, weight):
    B = indices.size
    V, _ = weight.shape
    idx = indices.reshape(-1).astype(jnp.int32)
    out = _build_lookup(B, V)(idx, weight)
    out = out.reshape(R, 8, NW, 8, BL).transpose(2, 4, 0, 1, 3)
    return out.reshape(*indices.shape, D)
